# Initial kernel scaffold; baseline (speedup 1.0000x reference)
import jax
import jax.numpy as jnp
from jax.experimental import pallas as pl
from jax.experimental.pallas import tpu as pltpu

_TEMP = 0.05
_B = 1024
_D = 64
_N = 100000
_BN = 2000
_GRID = _N // _BN


def _ce_kernel(x_ref, f_ref, t_ref, out_ref, m_ref, s_ref, g_ref):
    i = pl.program_id(0)

    @pl.when(i == 0)
    def _init():
        m_ref[...] = jnp.full_like(m_ref, -jnp.inf)
        s_ref[...] = jnp.zeros_like(s_ref)
        g_ref[...] = jnp.zeros_like(g_ref)

    x = x_ref[...]
    f = f_ref[...]
    z = jax.lax.dot_general(
        x, f, (((1,), (1,)), ((), ())), preferred_element_type=jnp.float32
    )
    mb = jnp.max(z, axis=1, keepdims=True)
    m_old = m_ref[...]
    m_new = jnp.maximum(m_old, mb)
    s_ref[...] = s_ref[...] * jnp.exp(m_old - m_new) + jnp.sum(
        jnp.exp(z - m_new), axis=1, keepdims=True
    )
    m_ref[...] = m_new

    col = jax.lax.broadcasted_iota(jnp.int32, z.shape, 1) + i * _BN
    hit = col == t_ref[...]
    g_ref[...] += jnp.sum(jnp.where(hit, z, 0.0), axis=1, keepdims=True)

    @pl.when(i == _GRID - 1)
    def _fin():
        lse = m_ref[...] + jnp.log(s_ref[...])
        out_ref[0, 0] = jnp.sum(lse - g_ref[...]) * (1.0 / _B)


def kernel(inputs, features, targets):
    x = inputs * (1.0 / _TEMP)
    t = targets.astype(jnp.int32).reshape(_B, 1)
    out = pl.pallas_call(
        _ce_kernel,
        grid=(_GRID,),
        in_specs=[
            pl.BlockSpec((_B, _D), lambda i: (0, 0)),
            pl.BlockSpec((_BN, _D), lambda i: (i, 0)),
            pl.BlockSpec((_B, 1), lambda i: (0, 0)),
        ],
        out_specs=pl.BlockSpec((1, 1), lambda i: (0, 0)),
        out_shape=jax.ShapeDtypeStruct((1, 1), jnp.float32),
        scratch_shapes=[
            pltpu.VMEM((_B, 1), jnp.float32),
            pltpu.VMEM((_B, 1), jnp.float32),
            pltpu.VMEM((_B, 1), jnp.float32),
        ],
    )(x, features, t)
    return out[0, 0]


# speedup vs baseline: 1.2769x; 1.2769x over previous
import jax
import jax.numpy as jnp
from jax.experimental import pallas as pl
from jax.experimental.pallas import tpu as pltpu

_TEMP = 0.05
_B = 1024
_D = 64
_N = 100000
_BN = 2000
_GRID = _N // _BN


def _ce_kernel(x_ref, f_ref, t_ref, out_ref, m_ref, s_ref, g_ref):
    i = pl.program_id(0)

    @pl.when(i == 0)
    def _init():
        m_ref[...] = jnp.full_like(m_ref, -jnp.inf)
        s_ref[...] = jnp.zeros_like(s_ref)
        g_ref[...] = jnp.zeros_like(g_ref)

    x = x_ref[...]
    f = f_ref[...]
    z = jax.lax.dot_general(
        x, f, (((1,), (1,)), ((), ())), preferred_element_type=jnp.float32
    )
    mb = jnp.max(z, axis=1, keepdims=True)
    m_old = m_ref[...]
    m_new = jnp.maximum(m_old, mb)
    s_ref[...] = s_ref[...] * jnp.exp(m_old - m_new) + jnp.sum(
        jnp.exp(z - m_new), axis=1, keepdims=True
    )
    m_ref[...] = m_new

    col = jax.lax.broadcasted_iota(jnp.int32, z.shape, 1) + i * _BN
    hit = col == t_ref[...]
    g_ref[...] += jnp.sum(jnp.where(hit, z, 0.0), axis=1, keepdims=True)

    @pl.when(i == _GRID - 1)
    def _fin():
        lse = m_ref[...] + jnp.log(s_ref[...])
        out_ref[...] = jnp.sum(lse - g_ref[...], keepdims=True) * (1.0 / _B)


def kernel(inputs, features, targets):
    x = inputs * (1.0 / _TEMP)
    t = targets.astype(jnp.int32).reshape(_B, 1)
    out = pl.pallas_call(
        _ce_kernel,
        grid=(_GRID,),
        in_specs=[
            pl.BlockSpec((_B, _D), lambda i: (0, 0)),
            pl.BlockSpec((_BN, _D), lambda i: (i, 0)),
            pl.BlockSpec((_B, 1), lambda i: (0, 0)),
        ],
        out_specs=pl.BlockSpec((1, 1), lambda i: (0, 0)),
        out_shape=jax.ShapeDtypeStruct((1, 1), jnp.float32),
        scratch_shapes=[
            pltpu.VMEM((_B, 1), jnp.float32),
            pltpu.VMEM((_B, 1), jnp.float32),
            pltpu.VMEM((_B, 1), jnp.float32),
        ],
    )(x, features, t)
    return out[0, 0]


# speedup vs baseline: 1.8238x; 1.4282x over previous
import jax
import jax.numpy as jnp
from jax.experimental import pallas as pl
from jax.experimental.pallas import tpu as pltpu

_TEMP = 0.05
_B = 1024
_D = 64
_N = 100000
_BN = 2000
_GRID = _N // _BN
_LOG2E = 1.4426950408889634
_LN2 = 0.6931471805599453
_C2 = 84.0


def _ce_kernel(x_ref, f_ref, t_ref, out_ref, mc_ref, s_ref, g_ref):
    i = pl.program_id(0)

    @pl.when(i == 0)
    def _init():
        x2 = x_ref[...]
        m2 = jnp.sqrt(jnp.sum(x2 * x2, axis=1, keepdims=True))
        mc_ref[...] = m2 - _C2
        s_ref[...] = jnp.zeros_like(s_ref)
        g_ref[...] = jnp.zeros_like(g_ref)

    z = jax.lax.dot_general(
        x_ref[...], f_ref[...], (((1,), (1,)), ((), ())),
        preferred_element_type=jnp.float32,
    )
    e = jnp.exp2(z - mc_ref[...])
    s_ref[...] += jnp.sum(e, axis=1, keepdims=True)

    col = jax.lax.broadcasted_iota(jnp.int32, z.shape, 1) + i * _BN
    hit = col == t_ref[...]
    g_ref[...] += jnp.sum(jnp.where(hit, z, 0.0), axis=1, keepdims=True)

    @pl.when(i == _GRID - 1)
    def _fin():
        lse2 = mc_ref[...] + jnp.log2(s_ref[...])
        out_ref[...] = jnp.sum(lse2 - g_ref[...], keepdims=True) * (_LN2 / _B)


def kernel(inputs, features, targets):
    x = inputs * (_LOG2E / _TEMP)
    t = targets.astype(jnp.int32).reshape(_B, 1)
    out = pl.pallas_call(
        _ce_kernel,
        grid=(_GRID,),
        in_specs=[
            pl.BlockSpec((_B, _D), lambda i: (0, 0)),
            pl.BlockSpec((_BN, _D), lambda i: (i, 0)),
            pl.BlockSpec((_B, 1), lambda i: (0, 0)),
        ],
        out_specs=pl.BlockSpec((1, 1), lambda i: (0, 0)),
        out_shape=jax.ShapeDtypeStruct((1, 1), jnp.float32),
        scratch_shapes=[
            pltpu.VMEM((_B, 1), jnp.float32),
            pltpu.VMEM((_B, 1), jnp.float32),
            pltpu.VMEM((_B, 1), jnp.float32),
        ],
    )(x, features, t)
    return out[0, 0]
